# baseline (speedup 1.0000x reference)
import math

import numpy as np
import jax
import jax.numpy as jnp
from jax import lax
from jax.experimental import pallas as pl
from jax.experimental.pallas import tpu as pltpu
from jax.experimental.pallas import tpu_sc as plsc

B = 4
S = 2048
D = 1024
BASE = D // 3
V = 128
SCALE = math.sqrt(D)

NC, NS = 2, 16
NW = NC * NS
SPW = S // NW
CH = 16
NQ = SPW // CH
SCB = 2
NCHUNK = NQ * SCB
NBUF = 5

TB = 256


def _pe_np() -> np.ndarray:
    p = np.arange(S, dtype=np.float64)[:, None]
    i = np.arange(D)
    i_even = ((i // 2) * 2).astype(np.float64)
    angle = p / np.power(10000.0, i_even / D)
    pe = np.where(i % 2 == 0, np.sin(angle), np.cos(angle))
    return pe.astype(np.float32)


_PE = _pe_np()


def _fused_body(cw_ref, sw_ref, qw_ref, sp_ref, pe_ref, out_ref):
    g = pl.program_id(0)

    @pl.when(g == 0)
    def _build_table():
        r = lax.broadcasted_iota(jnp.int32, (V, 8), 0)
        col = lax.broadcasted_iota(jnp.int32, (V, 8), 1)
        c = jnp.clip(r // 16, 0, 3)
        s = jnp.clip((r % 16) // 4, 0, 3)
        q = jnp.clip(r % 4, 0, 3)
        f32 = jnp.float32
        hi = lax.Precision.HIGHEST
        comp = jnp.concatenate(
            [
                lax.dot((col[:, :4] == c[:, :4]).astype(f32), cw_ref[:], precision=hi),
                lax.dot((col[:, :4] == s[:, :4]).astype(f32), sw_ref[:], precision=hi),
                lax.dot((col[:, :4] == q[:, :4]).astype(f32), qw_ref[:], precision=hi),
            ],
            axis=1,
        )
        spec = lax.dot(
            (col[:, :6] == jnp.clip(r[:, :6] - 64, 0, 5)).astype(f32),
            sp_ref[:],
            precision=hi,
        )
        is_comp = lax.broadcasted_iota(jnp.int32, (V, D), 0) < 64
        out_ref[:] = jnp.where(is_comp, comp, spec) * SCALE

    @pl.when(g > 0)
    def _copy_pe():
        out_ref[:] = pe_ref[:]


def _sc_body(
    fused_hbm, tok_hbm, out_hbm,
    idx_all, pe0, pe1, rows0, rows1, rows2, rows3, rows4,
    isem, ps0, ps1, gs0, gs1, gs2, gs3, gs4, os0, os1, os2, os3, os4,
):
    cid = lax.axis_index("c")
    sid = lax.axis_index("s")
    w = sid * NC + cid
    s0 = w * SPW
    pe = (pe0, pe1)
    rows = (rows0, rows1, rows2, rows3, rows4)
    gsem = (gs0, gs1, gs2, gs3, gs4)
    osem = (os0, os1, os2, os3, os4)
    psem = (ps0, ps1)

    def rbase(i):
        return (i % SCB) * S + s0 + (i // SCB) * CH

    def start_gather(i):
        p = i % NBUF
        off = (i % SCB) * SPW + (i // SCB) * CH
        return pltpu.async_copy(
            fused_hbm.at[idx_all.at[pl.ds(off, CH)]], rows[p], gsem[p]
        )

    def start_pe(h):
        return pltpu.async_copy(
            fused_hbm.at[pl.ds(V + s0 + h * CH, CH)], pe[h % 2], psem[h % 2]
        )

    gdesc = [None] * NCHUNK
    odesc = [None] * NCHUNK
    pdesc = [None] * NQ

    idesc = [
        pltpu.async_copy(
            tok_hbm.at[pl.ds(b * S + s0, SPW)], idx_all.at[pl.ds(b * SPW, SPW)], isem
        )
        for b in range(SCB)
    ]
    pdesc[0] = start_pe(0)
    for d in idesc:
        d.wait()
    for i in range(NBUF - 1):
        gdesc[i] = start_gather(i)

    for i in range(NCHUNK):
        p = i % NBUF
        h = i // SCB
        if i % SCB == 0:
            pdesc[h].wait()
            if h + 1 < NQ:
                pdesc[h + 1] = start_pe(h + 1)
        gdesc[i].wait()
        pe_v = pe[h % 2]

        def _row(r, carry, rows_v=rows[p], pe_v=pe_v):
            for j in range(D // 16):
                plsc.addupdate(
                    rows_v.at[r, pl.ds(j * 16, 16)],
                    pe_v[r, pl.ds(j * 16, 16)],
                )
            return carry

        lax.fori_loop(0, CH, _row, 0)
        odesc[i] = pltpu.async_copy(rows[p], out_hbm.at[pl.ds(rbase(i), CH)], osem[p])
        j = i + NBUF - 1
        if j < NCHUNK:
            if j >= NBUF:
                odesc[j - NBUF].wait()
            gdesc[j] = start_gather(j)
    for i in range(NCHUNK - NBUF, NCHUNK):
        odesc[i].wait()


def _embed_body(acc_ref, tok_ref, table_ref, pe_ref, out_ref):
    del acc_ref
    tokb = tok_ref[:]
    oh = (
        tokb[:, None] == lax.broadcasted_iota(jnp.int32, (TB, V), 1)
    ).astype(jnp.float32)
    emb = lax.dot(oh, table_ref[:], precision=lax.Precision.HIGHEST)
    out_ref[0] = emb + pe_ref[:]


def kernel(input_x, color_W, shape_W, quantity_W, special_W):
    f32 = jnp.float32
    npe = S // V
    fused = pl.pallas_call(
        _fused_body,
        grid=(1 + npe,),
        in_specs=[
            pl.BlockSpec((4, BASE), lambda g: (0, 0)),
            pl.BlockSpec((4, BASE), lambda g: (0, 0)),
            pl.BlockSpec((4, D - 2 * BASE), lambda g: (0, 0)),
            pl.BlockSpec((6, D), lambda g: (0, 0)),
            pl.BlockSpec((V, D), lambda g: (jnp.maximum(g - 1, 0), 0)),
        ],
        out_specs=pl.BlockSpec((V, D), lambda g: (g, 0)),
        out_shape=jax.ShapeDtypeStruct((V + S, D), f32),
    )(color_W, shape_W, quantity_W, special_W, jnp.asarray(_PE))

    tok = input_x.astype(jnp.int32).reshape(-1)

    mesh = plsc.VectorSubcoreMesh(core_axis_name="c", subcore_axis_name="s")
    sc_out = pl.kernel(
        _sc_body,
        out_type=jax.ShapeDtypeStruct((B * S, D), f32),
        mesh=mesh,
        scratch_types=(
            [pltpu.VMEM((SCB * SPW,), jnp.int32)]
            + [pltpu.VMEM((CH, D), f32)] * 2
            + [pltpu.VMEM((CH, D), f32)] * NBUF
            + [pltpu.SemaphoreType.DMA] * (2 * NBUF + 3)
        ),
    )(fused, tok)

    out = pl.pallas_call(
        _embed_body,
        grid=(S // TB, B - SCB),
        in_specs=[
            pl.BlockSpec(memory_space=pltpu.MemorySpace.HBM),
            pl.BlockSpec((TB,), lambda si, bi: ((SCB + bi) * (S // TB) + si,)),
            pl.BlockSpec((V, D), lambda si, bi: (0, 0)),
            pl.BlockSpec((TB, D), lambda si, bi: (si, 0)),
        ],
        out_specs=pl.BlockSpec((1, TB, D), lambda si, bi: (SCB + bi, si, 0)),
        out_shape=jax.ShapeDtypeStruct((B, S, D), f32),
        input_output_aliases={0: 0},
    )(sc_out.reshape(B, S, D), tok, fused, jnp.asarray(_PE))
    return out


# speedup vs baseline: 1.1130x; 1.1130x over previous
import math

import numpy as np
import jax
import jax.numpy as jnp
from jax import lax
from jax.experimental import pallas as pl
from jax.experimental.pallas import tpu as pltpu
from jax.experimental.pallas import tpu_sc as plsc

B = 4
S = 2048
D = 1024
BASE = D // 3
V = 128
SCALE = math.sqrt(D)

NC, NS = 2, 16
NW = NC * NS
SPW = S // NW
CH = 16
NQ = SPW // CH
SCB = 2
NCHUNK = NQ * SCB
NBUF = 5

TB = 256


def _pe_np() -> np.ndarray:
    p = np.arange(S, dtype=np.float64)[:, None]
    i = np.arange(D)
    i_even = ((i // 2) * 2).astype(np.float64)
    angle = p / np.power(10000.0, i_even / D)
    pe = np.where(i % 2 == 0, np.sin(angle), np.cos(angle))
    return pe.astype(np.float32)


_PE = _pe_np()


def _fused_body(cw_ref, sw_ref, qw_ref, sp_ref, pe_ref, out_ref):
    r = lax.broadcasted_iota(jnp.int32, (V, 8), 0)
    col = lax.broadcasted_iota(jnp.int32, (V, 8), 1)
    c = jnp.clip(r // 16, 0, 3)
    s = jnp.clip((r % 16) // 4, 0, 3)
    q = jnp.clip(r % 4, 0, 3)
    f32 = jnp.float32
    hi = lax.Precision.HIGHEST
    comp = jnp.concatenate(
        [
            lax.dot((col[:, :4] == c[:, :4]).astype(f32), cw_ref[:], precision=hi),
            lax.dot((col[:, :4] == s[:, :4]).astype(f32), sw_ref[:], precision=hi),
            lax.dot((col[:, :4] == q[:, :4]).astype(f32), qw_ref[:], precision=hi),
        ],
        axis=1,
    )
    spec = lax.dot(
        (col[:, :6] == jnp.clip(r[:, :6] - 64, 0, 5)).astype(f32),
        sp_ref[:],
        precision=hi,
    )
    is_comp = lax.broadcasted_iota(jnp.int32, (V, D), 0) < 64
    out_ref[:V] = jnp.where(is_comp, comp, spec) * SCALE
    out_ref[V:] = pe_ref[:]


def _sc_body(
    fused_hbm, tok_hbm, out_hbm,
    idx_all, pe0, pe1, rows0, rows1, rows2, rows3, rows4,
    isem, ps0, ps1, gs0, gs1, gs2, gs3, gs4, os0, os1, os2, os3, os4,
):
    cid = lax.axis_index("c")
    sid = lax.axis_index("s")
    w = sid * NC + cid
    s0 = w * SPW
    pe = (pe0, pe1)
    rows = (rows0, rows1, rows2, rows3, rows4)
    gsem = (gs0, gs1, gs2, gs3, gs4)
    osem = (os0, os1, os2, os3, os4)
    psem = (ps0, ps1)

    def rbase(i):
        return (i % SCB) * S + s0 + (i // SCB) * CH

    def start_gather(i):
        p = i % NBUF
        off = (i % SCB) * SPW + (i // SCB) * CH
        return pltpu.async_copy(
            fused_hbm.at[idx_all.at[pl.ds(off, CH)]], rows[p], gsem[p]
        )

    def start_pe(h):
        return pltpu.async_copy(
            fused_hbm.at[pl.ds(V + s0 + h * CH, CH)], pe[h % 2], psem[h % 2]
        )

    gdesc = [None] * NCHUNK
    odesc = [None] * NCHUNK
    pdesc = [None] * NQ

    idesc = [
        pltpu.async_copy(
            tok_hbm.at[pl.ds(b * S + s0, SPW)], idx_all.at[pl.ds(b * SPW, SPW)], isem
        )
        for b in range(SCB)
    ]
    pdesc[0] = start_pe(0)
    for d in idesc:
        d.wait()
    for i in range(NBUF - 1):
        gdesc[i] = start_gather(i)

    for i in range(NCHUNK):
        p = i % NBUF
        h = i // SCB
        if i % SCB == 0:
            pdesc[h].wait()
            if h + 1 < NQ:
                pdesc[h + 1] = start_pe(h + 1)
        gdesc[i].wait()
        pe_v = pe[h % 2]

        def _row(r, carry, rows_v=rows[p], pe_v=pe_v):
            for j in range(D // 16):
                plsc.addupdate(
                    rows_v.at[r, pl.ds(j * 16, 16)],
                    pe_v[r, pl.ds(j * 16, 16)],
                )
            return carry

        lax.fori_loop(0, CH, _row, 0)
        odesc[i] = pltpu.async_copy(rows[p], out_hbm.at[pl.ds(rbase(i), CH)], osem[p])
        j = i + NBUF - 1
        if j < NCHUNK:
            if j >= NBUF:
                odesc[j - NBUF].wait()
            gdesc[j] = start_gather(j)
    for i in range(NCHUNK - NBUF, NCHUNK):
        odesc[i].wait()


def _embed_body(acc_ref, tok_ref, table_ref, pe_ref, out_ref):
    del acc_ref
    bf16, f32 = jnp.bfloat16, jnp.float32
    tokb = tok_ref[:]
    oh = (
        tokb[:, None] == lax.broadcasted_iota(jnp.int32, (TB, V), 1)
    ).astype(bf16)
    tab = table_ref[:]
    th = tab.astype(bf16)
    tl = (tab - th.astype(f32)).astype(bf16)
    emb = lax.dot(oh, th, preferred_element_type=f32) + lax.dot(
        oh, tl, preferred_element_type=f32
    )
    out_ref[0] = emb + pe_ref[:]


def kernel(input_x, color_W, shape_W, quantity_W, special_W):
    f32 = jnp.float32
    fused = pl.pallas_call(
        _fused_body,
        out_shape=jax.ShapeDtypeStruct((V + S, D), f32),
    )(color_W, shape_W, quantity_W, special_W, jnp.asarray(_PE))

    tok = input_x.astype(jnp.int32).reshape(-1)

    mesh = plsc.VectorSubcoreMesh(core_axis_name="c", subcore_axis_name="s")
    sc_out = pl.kernel(
        _sc_body,
        out_type=jax.ShapeDtypeStruct((B * S, D), f32),
        mesh=mesh,
        scratch_types=(
            [pltpu.VMEM((SCB * SPW,), jnp.int32)]
            + [pltpu.VMEM((CH, D), f32)] * 2
            + [pltpu.VMEM((CH, D), f32)] * NBUF
            + [pltpu.SemaphoreType.DMA] * (2 * NBUF + 3)
        ),
    )(fused, tok)

    out = pl.pallas_call(
        _embed_body,
        grid=(S // TB, B - SCB),
        in_specs=[
            pl.BlockSpec(memory_space=pltpu.MemorySpace.HBM),
            pl.BlockSpec((TB,), lambda si, bi: ((SCB + bi) * (S // TB) + si,)),
            pl.BlockSpec((V, D), lambda si, bi: (0, 0)),
            pl.BlockSpec((TB, D), lambda si, bi: (si, 0)),
        ],
        out_specs=pl.BlockSpec((1, TB, D), lambda si, bi: (SCB + bi, si, 0)),
        out_shape=jax.ShapeDtypeStruct((B, S, D), f32),
        input_output_aliases={0: 0},
    )(sc_out.reshape(B, S, D), tok, fused, jnp.asarray(_PE))
    return out


# speedup vs baseline: 1.2200x; 1.0962x over previous
import math

import numpy as np
import jax
import jax.numpy as jnp
from jax import lax
from jax.experimental import pallas as pl
from jax.experimental.pallas import tpu as pltpu
from jax.experimental.pallas import tpu_sc as plsc

B = 4
S = 2048
D = 1024
BASE = D // 3
V = 128
SCALE = math.sqrt(D)

NC, NS = 2, 16
NW = NC * NS
SPW = S // NW
CH = 16
NQ = SPW // CH
SCB = 2
NCHUNK = NQ * SCB
NBUF = 5

TB = 512


def _pe_np() -> np.ndarray:
    p = np.arange(S, dtype=np.float64)[:, None]
    i = np.arange(D)
    i_even = ((i // 2) * 2).astype(np.float64)
    angle = p / np.power(10000.0, i_even / D)
    pe = np.where(i % 2 == 0, np.sin(angle), np.cos(angle))
    return pe.astype(np.float32)


_PE = _pe_np()


def _fused_body(cw_ref, sw_ref, qw_ref, sp_ref, pe_ref, out_ref):
    r = lax.broadcasted_iota(jnp.int32, (V, 8), 0)
    col = lax.broadcasted_iota(jnp.int32, (V, 8), 1)
    c = jnp.clip(r // 16, 0, 3)
    s = jnp.clip((r % 16) // 4, 0, 3)
    q = jnp.clip(r % 4, 0, 3)
    f32 = jnp.float32
    hi = lax.Precision.HIGHEST
    comp = jnp.concatenate(
        [
            lax.dot((col[:, :4] == c[:, :4]).astype(f32), cw_ref[:], precision=hi),
            lax.dot((col[:, :4] == s[:, :4]).astype(f32), sw_ref[:], precision=hi),
            lax.dot((col[:, :4] == q[:, :4]).astype(f32), qw_ref[:], precision=hi),
        ],
        axis=1,
    )
    spec = lax.dot(
        (col[:, :6] == jnp.clip(r[:, :6] - 64, 0, 5)).astype(f32),
        sp_ref[:],
        precision=hi,
    )
    is_comp = lax.broadcasted_iota(jnp.int32, (V, D), 0) < 64
    out_ref[:V] = jnp.where(is_comp, comp, spec) * SCALE
    out_ref[V:] = pe_ref[:]


def _sc_body(
    fused_hbm, tok_hbm, out_hbm,
    idx_all, pe0, pe1, rows0, rows1, rows2, rows3, rows4,
    isem, ps0, ps1, gs0, gs1, gs2, gs3, gs4, os0, os1, os2, os3, os4,
):
    cid = lax.axis_index("c")
    sid = lax.axis_index("s")
    w = sid * NC + cid
    s0 = w * SPW
    pe = (pe0, pe1)
    rows = (rows0, rows1, rows2, rows3, rows4)
    gsem = (gs0, gs1, gs2, gs3, gs4)
    osem = (os0, os1, os2, os3, os4)
    psem = (ps0, ps1)

    def rbase(i):
        return (i % SCB) * S + s0 + (i // SCB) * CH

    def start_gather(i):
        p = i % NBUF
        off = (i % SCB) * SPW + (i // SCB) * CH
        return pltpu.async_copy(
            fused_hbm.at[idx_all.at[pl.ds(off, CH)]], rows[p], gsem[p]
        )

    def start_pe(h):
        return pltpu.async_copy(
            fused_hbm.at[pl.ds(V + s0 + h * CH, CH)], pe[h % 2], psem[h % 2]
        )

    gdesc = [None] * NCHUNK
    odesc = [None] * NCHUNK
    pdesc = [None] * NQ

    idesc = [
        pltpu.async_copy(
            tok_hbm.at[pl.ds(b * S + s0, SPW)], idx_all.at[pl.ds(b * SPW, SPW)], isem
        )
        for b in range(SCB)
    ]
    pdesc[0] = start_pe(0)
    for d in idesc:
        d.wait()
    for i in range(NBUF - 1):
        gdesc[i] = start_gather(i)

    for i in range(NCHUNK):
        p = i % NBUF
        h = i // SCB
        if i % SCB == 0:
            pdesc[h].wait()
            if h + 1 < NQ:
                pdesc[h + 1] = start_pe(h + 1)
        gdesc[i].wait()
        pe_v = pe[h % 2]

        def _row(r, carry, rows_v=rows[p], pe_v=pe_v):
            for j in range(D // 16):
                plsc.addupdate(
                    rows_v.at[r, pl.ds(j * 16, 16)],
                    pe_v[r, pl.ds(j * 16, 16)],
                )
            return carry

        lax.fori_loop(0, CH, _row, 0)
        odesc[i] = pltpu.async_copy(rows[p], out_hbm.at[pl.ds(rbase(i), CH)], osem[p])
        j = i + NBUF - 1
        if j < NCHUNK:
            if j >= NBUF:
                odesc[j - NBUF].wait()
            gdesc[j] = start_gather(j)
    for i in range(NCHUNK - NBUF, NCHUNK):
        odesc[i].wait()


def _embed_body(acc_ref, tok_ref, table_ref, pe_ref, out_ref):
    del acc_ref
    bf16, f32 = jnp.bfloat16, jnp.float32
    tokb = tok_ref[:]
    oh = (
        tokb[:, None] == lax.broadcasted_iota(jnp.int32, (TB, V), 1)
    ).astype(bf16)
    emb = lax.dot(oh, table_ref[:].astype(bf16), preferred_element_type=f32)
    out_ref[0] = emb + pe_ref[:]


def kernel(input_x, color_W, shape_W, quantity_W, special_W):
    f32 = jnp.float32
    fused = pl.pallas_call(
        _fused_body,
        out_shape=jax.ShapeDtypeStruct((V + S, D), f32),
    )(color_W, shape_W, quantity_W, special_W, jnp.asarray(_PE))

    tok = input_x.astype(jnp.int32).reshape(-1)

    mesh = plsc.VectorSubcoreMesh(core_axis_name="c", subcore_axis_name="s")
    sc_out = pl.kernel(
        _sc_body,
        out_type=jax.ShapeDtypeStruct((B * S, D), f32),
        mesh=mesh,
        scratch_types=(
            [pltpu.VMEM((SCB * SPW,), jnp.int32)]
            + [pltpu.VMEM((CH, D), f32)] * 2
            + [pltpu.VMEM((CH, D), f32)] * NBUF
            + [pltpu.SemaphoreType.DMA] * (2 * NBUF + 3)
        ),
    )(fused, tok)

    out = pl.pallas_call(
        _embed_body,
        grid=(S // TB, B - SCB),
        in_specs=[
            pl.BlockSpec(memory_space=pltpu.MemorySpace.HBM),
            pl.BlockSpec((TB,), lambda si, bi: ((SCB + bi) * (S // TB) + si,)),
            pl.BlockSpec((V, D), lambda si, bi: (0, 0)),
            pl.BlockSpec((TB, D), lambda si, bi: (si, 0)),
        ],
        out_specs=pl.BlockSpec((1, TB, D), lambda si, bi: (SCB + bi, si, 0)),
        out_shape=jax.ShapeDtypeStruct((B, S, D), f32),
        input_output_aliases={0: 0},
    )(sc_out.reshape(B, S, D), tok, fused, jnp.asarray(_PE))
    return out


# speedup vs baseline: 1.2507x; 1.0252x over previous
import math

import numpy as np
import jax
import jax.numpy as jnp
from jax import lax
from jax.experimental import pallas as pl
from jax.experimental.pallas import tpu as pltpu
from jax.experimental.pallas import tpu_sc as plsc

B = 4
S = 2048
D = 1024
BASE = D // 3
V = 128
SCALE = math.sqrt(D)

NC, NS = 2, 16
NW = NC * NS
SPW = S // NW
CH = 16
NQ = SPW // CH
SCB = 2
NCHUNK = NQ * SCB
NBUF = 5

TB = 512


def _pe_np() -> np.ndarray:
    p = np.arange(S, dtype=np.float64)[:, None]
    i = np.arange(D)
    i_even = ((i // 2) * 2).astype(np.float64)
    angle = p / np.power(10000.0, i_even / D)
    pe = np.where(i % 2 == 0, np.sin(angle), np.cos(angle))
    return pe.astype(np.float32)


_PE = _pe_np()


def _fused_body(cw_ref, sw_ref, qw_ref, sp_ref, pe_ref, tok_ref, out_ref, tok_out_ref):
    r = lax.broadcasted_iota(jnp.int32, (V, 8), 0)
    col = lax.broadcasted_iota(jnp.int32, (V, 8), 1)
    c = jnp.clip(r // 16, 0, 3)
    s = jnp.clip((r % 16) // 4, 0, 3)
    q = jnp.clip(r % 4, 0, 3)
    f32 = jnp.float32
    hi = lax.Precision.HIGHEST
    comp = jnp.concatenate(
        [
            lax.dot((col[:, :4] == c[:, :4]).astype(f32), cw_ref[:], precision=hi),
            lax.dot((col[:, :4] == s[:, :4]).astype(f32), sw_ref[:], precision=hi),
            lax.dot((col[:, :4] == q[:, :4]).astype(f32), qw_ref[:], precision=hi),
        ],
        axis=1,
    )
    spec = lax.dot(
        (col[:, :6] == jnp.clip(r[:, :6] - 64, 0, 5)).astype(f32),
        sp_ref[:],
        precision=hi,
    )
    is_comp = lax.broadcasted_iota(jnp.int32, (V, D), 0) < 64
    out_ref[:V] = jnp.where(is_comp, comp, spec) * SCALE
    out_ref[V:] = pe_ref[:]
    tok_out_ref[:] = tok_ref[:]


def _sc_body(
    fused_hbm, tok_hbm, out_hbm,
    idx_all, pe0, pe1, rows0, rows1, rows2, rows3, rows4,
    isem, ps0, ps1, gs0, gs1, gs2, gs3, gs4, os0, os1, os2, os3, os4,
):
    cid = lax.axis_index("c")
    sid = lax.axis_index("s")
    w = sid * NC + cid
    s0 = w * SPW
    pe = (pe0, pe1)
    rows = (rows0, rows1, rows2, rows3, rows4)
    gsem = (gs0, gs1, gs2, gs3, gs4)
    osem = (os0, os1, os2, os3, os4)
    psem = (ps0, ps1)

    def rbase(i):
        return (i % SCB) * S + s0 + (i // SCB) * CH

    def start_gather(i):
        p = i % NBUF
        off = (i % SCB) * SPW + (i // SCB) * CH
        return pltpu.async_copy(
            fused_hbm.at[idx_all.at[pl.ds(off, CH)]], rows[p], gsem[p]
        )

    def start_pe(h):
        return pltpu.async_copy(
            fused_hbm.at[pl.ds(V + s0 + h * CH, CH)], pe[h % 2], psem[h % 2]
        )

    gdesc = [None] * NCHUNK
    odesc = [None] * NCHUNK
    pdesc = [None] * NQ

    idesc = [
        pltpu.async_copy(
            tok_hbm.at[b, pl.ds(s0, SPW)], idx_all.at[pl.ds(b * SPW, SPW)], isem
        )
        for b in range(SCB)
    ]
    pdesc[0] = start_pe(0)
    for d in idesc:
        d.wait()
    for i in range(NBUF - 1):
        gdesc[i] = start_gather(i)

    for i in range(NCHUNK):
        p = i % NBUF
        h = i // SCB
        if i % SCB == 0:
            pdesc[h].wait()
            if h + 1 < NQ:
                pdesc[h + 1] = start_pe(h + 1)
        gdesc[i].wait()
        pe_v = pe[h % 2]

        def _row(r, carry, rows_v=rows[p], pe_v=pe_v):
            for j in range(D // 16):
                plsc.addupdate(
                    rows_v.at[r, pl.ds(j * 16, 16)],
                    pe_v[r, pl.ds(j * 16, 16)],
                )
            return carry

        lax.fori_loop(0, CH, _row, 0)
        odesc[i] = pltpu.async_copy(rows[p], out_hbm.at[pl.ds(rbase(i), CH)], osem[p])
        j = i + NBUF - 1
        if j < NCHUNK:
            if j >= NBUF:
                odesc[j - NBUF].wait()
            gdesc[j] = start_gather(j)
    for i in range(NCHUNK - NBUF, NCHUNK):
        odesc[i].wait()


def _embed_body(acc_ref, tok_ref, table_ref, pe_ref, out_ref):
    del acc_ref
    bf16, f32 = jnp.bfloat16, jnp.float32
    tokb = tok_ref[:]
    oh = (
        tokb[:, None] == lax.broadcasted_iota(jnp.int32, (TB, V), 1)
    ).astype(bf16)
    emb = lax.dot(oh, table_ref[:].astype(bf16), preferred_element_type=f32)
    out_ref[0] = emb + pe_ref[:]


def kernel(input_x, color_W, shape_W, quantity_W, special_W):
    f32 = jnp.float32
    tok2d = input_x.astype(jnp.int32)
    fused, tok_sc = pl.pallas_call(
        _fused_body,
        out_shape=(
            jax.ShapeDtypeStruct((V + S, D), f32),
            jax.ShapeDtypeStruct((B, S), jnp.int32),
        ),
    )(color_W, shape_W, quantity_W, special_W, jnp.asarray(_PE), tok2d)

    tok = tok2d.reshape(-1)

    mesh = plsc.VectorSubcoreMesh(core_axis_name="c", subcore_axis_name="s")
    sc_out = pl.kernel(
        _sc_body,
        out_type=jax.ShapeDtypeStruct((B * S, D), f32),
        mesh=mesh,
        scratch_types=(
            [pltpu.VMEM((SCB * SPW,), jnp.int32)]
            + [pltpu.VMEM((CH, D), f32)] * 2
            + [pltpu.VMEM((CH, D), f32)] * NBUF
            + [pltpu.SemaphoreType.DMA] * (2 * NBUF + 3)
        ),
    )(fused, tok_sc)

    out = pl.pallas_call(
        _embed_body,
        grid=(S // TB, B - SCB),
        in_specs=[
            pl.BlockSpec(memory_space=pltpu.MemorySpace.HBM),
            pl.BlockSpec((TB,), lambda si, bi: ((SCB + bi) * (S // TB) + si,)),
            pl.BlockSpec((V, D), lambda si, bi: (0, 0)),
            pl.BlockSpec((TB, D), lambda si, bi: (si, 0)),
        ],
        out_specs=pl.BlockSpec((1, TB, D), lambda si, bi: (SCB + bi, si, 0)),
        out_shape=jax.ShapeDtypeStruct((B, S, D), f32),
        input_output_aliases={0: 0},
    )(sc_out.reshape(B, S, D), tok, fused, jnp.asarray(_PE))
    return out


# speedup vs baseline: 1.2520x; 1.0011x over previous
import math

import numpy as np
import jax
import jax.numpy as jnp
from jax import lax
from jax.experimental import pallas as pl
from jax.experimental.pallas import tpu as pltpu
from jax.experimental.pallas import tpu_sc as plsc

B = 4
S = 2048
D = 1024
BASE = D // 3
V = 128
SCALE = math.sqrt(D)

NC, NS = 2, 16
NW = NC * NS
SPW = S // NW
CH = 16
NQ = SPW // CH
SCB = 2
NCHUNK = NQ * SCB
NBUF = 5

TB = 512


def _pe_np() -> np.ndarray:
    p = np.arange(S, dtype=np.float64)[:, None]
    i = np.arange(D)
    i_even = ((i // 2) * 2).astype(np.float64)
    angle = p / np.power(10000.0, i_even / D)
    pe = np.where(i % 2 == 0, np.sin(angle), np.cos(angle))
    return pe.astype(np.float32)


_PE = _pe_np()


def _fused_body(cw_ref, sw_ref, qw_ref, sp_ref, pe_ref, tok_ref, out_ref, tok_out_ref):
    r = lax.broadcasted_iota(jnp.int32, (V, 8), 0)
    col = lax.broadcasted_iota(jnp.int32, (V, 8), 1)
    c = jnp.clip(r // 16, 0, 3)
    s = jnp.clip((r % 16) // 4, 0, 3)
    q = jnp.clip(r % 4, 0, 3)
    f32 = jnp.float32
    hi = lax.Precision.HIGHEST
    comp = jnp.concatenate(
        [
            lax.dot((col[:, :4] == c[:, :4]).astype(f32), cw_ref[:], precision=hi),
            lax.dot((col[:, :4] == s[:, :4]).astype(f32), sw_ref[:], precision=hi),
            lax.dot((col[:, :4] == q[:, :4]).astype(f32), qw_ref[:], precision=hi),
        ],
        axis=1,
    )
    spec = lax.dot(
        (col[:, :6] == jnp.clip(r[:, :6] - 64, 0, 5)).astype(f32),
        sp_ref[:],
        precision=hi,
    )
    is_comp = lax.broadcasted_iota(jnp.int32, (V, D), 0) < 64
    out_ref[:V] = jnp.where(is_comp, comp, spec) * SCALE
    out_ref[V:] = pe_ref[:]
    tok_out_ref[:] = tok_ref[:]


def _sc_body(
    fused_hbm, tok_hbm, out_hbm,
    idx_all, pe0, pe1, rows0, rows1, rows2, rows3, rows4,
    isem, ps0, ps1, gs0, gs1, gs2, gs3, gs4, os0, os1, os2, os3, os4,
):
    cid = lax.axis_index("c")
    sid = lax.axis_index("s")
    w = sid * NC + cid
    s0 = w * SPW
    pe = (pe0, pe1)
    rows = (rows0, rows1, rows2, rows3, rows4)
    gsem = (gs0, gs1, gs2, gs3, gs4)
    osem = (os0, os1, os2, os3, os4)
    psem = (ps0, ps1)

    def rbase(i):
        return (i % SCB) * S + s0 + (i // SCB) * CH

    def start_gather(i):
        p = i % NBUF
        off = (i % SCB) * SPW + (i // SCB) * CH
        return pltpu.async_copy(
            fused_hbm.at[idx_all.at[pl.ds(off, CH)]], rows[p], gsem[p]
        )

    def start_pe(h):
        return pltpu.async_copy(
            fused_hbm.at[pl.ds(V + s0 + h * CH, CH)], pe[h % 2], psem[h % 2]
        )

    gdesc = [None] * NCHUNK
    odesc = [None] * NCHUNK
    pdesc = [None] * NQ

    idesc = [
        pltpu.async_copy(
            tok_hbm.at[b, pl.ds(s0, SPW)], idx_all.at[pl.ds(b * SPW, SPW)], isem
        )
        for b in range(SCB)
    ]
    pdesc[0] = start_pe(0)
    for d in idesc:
        d.wait()
    for i in range(2 * SCB):
        gdesc[i] = start_gather(i)

    for h in range(NQ):
        i0, i1 = SCB * h, SCB * h + 1
        p0, p1 = i0 % NBUF, i1 % NBUF
        pdesc[h].wait()
        if h + 1 < NQ:
            pdesc[h + 1] = start_pe(h + 1)
        gdesc[i0].wait()
        gdesc[i1].wait()
        pe_v = pe[h % 2]

        def _row(r, carry, rows_a=rows[p0], rows_b=rows[p1], pe_v=pe_v):
            for j in range(D // 16):
                v = pe_v[r, pl.ds(j * 16, 16)]
                plsc.addupdate(rows_a.at[r, pl.ds(j * 16, 16)], v)
                plsc.addupdate(rows_b.at[r, pl.ds(j * 16, 16)], v)
            return carry

        lax.fori_loop(0, CH, _row, 0)
        odesc[i0] = pltpu.async_copy(rows[p0], out_hbm.at[pl.ds(rbase(i0), CH)], osem[p0])
        odesc[i1] = pltpu.async_copy(rows[p1], out_hbm.at[pl.ds(rbase(i1), CH)], osem[p1])
        for j in (i0 + 2 * SCB, i1 + 2 * SCB):
            if j < NCHUNK:
                if j >= NBUF:
                    odesc[j - NBUF].wait()
                gdesc[j] = start_gather(j)
    for i in range(NCHUNK - NBUF, NCHUNK):
        odesc[i].wait()


def _embed_body(acc_ref, tok_ref, table_ref, pe_ref, out_ref):
    del acc_ref
    bf16, f32 = jnp.bfloat16, jnp.float32
    tokb = tok_ref[:]
    oh = (
        tokb[:, None] == lax.broadcasted_iota(jnp.int32, (TB, V), 1)
    ).astype(bf16)
    emb = lax.dot(oh, table_ref[:].astype(bf16), preferred_element_type=f32)
    out_ref[0] = emb + pe_ref[:]


def kernel(input_x, color_W, shape_W, quantity_W, special_W):
    f32 = jnp.float32
    tok2d = input_x.astype(jnp.int32)
    fused, tok_sc = pl.pallas_call(
        _fused_body,
        out_shape=(
            jax.ShapeDtypeStruct((V + S, D), f32),
            jax.ShapeDtypeStruct((B, S), jnp.int32),
        ),
    )(color_W, shape_W, quantity_W, special_W, jnp.asarray(_PE), tok2d)

    tok = tok2d.reshape(-1)

    mesh = plsc.VectorSubcoreMesh(core_axis_name="c", subcore_axis_name="s")
    sc_out = pl.kernel(
        _sc_body,
        out_type=jax.ShapeDtypeStruct((B * S, D), f32),
        mesh=mesh,
        scratch_types=(
            [pltpu.VMEM((SCB * SPW,), jnp.int32)]
            + [pltpu.VMEM((CH, D), f32)] * 2
            + [pltpu.VMEM((CH, D), f32)] * NBUF
            + [pltpu.SemaphoreType.DMA] * (2 * NBUF + 3)
        ),
    )(fused, tok_sc)

    out = pl.pallas_call(
        _embed_body,
        grid=(S // TB, B - SCB),
        in_specs=[
            pl.BlockSpec(memory_space=pltpu.MemorySpace.HBM),
            pl.BlockSpec((TB,), lambda si, bi: ((SCB + bi) * (S // TB) + si,)),
            pl.BlockSpec((V, D), lambda si, bi: (0, 0)),
            pl.BlockSpec((TB, D), lambda si, bi: (si, 0)),
        ],
        out_specs=pl.BlockSpec((1, TB, D), lambda si, bi: (SCB + bi, si, 0)),
        out_shape=jax.ShapeDtypeStruct((B, S, D), f32),
        input_output_aliases={0: 0},
    )(sc_out.reshape(B, S, D), tok, fused, jnp.asarray(_PE))
    return out
